# baseline (speedup 1.0000x reference)
import jax
import jax.numpy as jnp
from jax.experimental import pallas as pl

_BLOCK_N = 128
_LOG2E = 1.4426950408889634


def _fused_kernel(q_ref, items_ref, o_ref):
    l, b, c = q_ref.shape
    s = jax.lax.dot_general(
        q_ref[...], items_ref[...],
        (((2,), (1,)), ((), ())),
        preferred_element_type=jnp.float32,
    )
    m = jnp.max(s, axis=2, keepdims=True)
    denom = jnp.sum(jnp.exp(s - m), axis=2)
    o_ref[...] = 1.0 / denom


@jax.jit
def kernel(queries, items):
    n, l, c = queries.shape
    m_items = items.shape[0]
    t = n * l
    qt = jnp.transpose(queries, (1, 0, 2))
    grid = (n // _BLOCK_N,)
    rows_per_step = _BLOCK_N * l // 128
    out = pl.pallas_call(
        _fused_kernel,
        grid=grid,
        in_specs=[
            pl.BlockSpec((l, _BLOCK_N, c), lambda i: (0, i, 0)),
            pl.BlockSpec((m_items, c), lambda i: (0, 0)),
        ],
        out_specs=pl.BlockSpec((l, _BLOCK_N), lambda i: (0, i)),
        out_shape=jax.ShapeDtypeStruct((l, n), jnp.float32),
    )(qt, items)
    return out.T.reshape(t, 1)


# speedup vs baseline: 1.0409x; 1.0409x over previous
import jax
import jax.numpy as jnp
from jax.experimental import pallas as pl

_BLOCK_N = 256
_LOG2E = 1.4426950408889634


def _fused_kernel(q_ref, items_ref, o_ref):
    l, b, c = q_ref.shape
    s = jax.lax.dot_general(
        q_ref[...], items_ref[...],
        (((2,), (1,)), ((), ())),
        preferred_element_type=jnp.float32,
    )
    m = jnp.max(s, axis=2, keepdims=True)
    denom = jnp.sum(jnp.exp(s - m), axis=2)
    o_ref[...] = 1.0 / denom


@jax.jit
def kernel(queries, items):
    n, l, c = queries.shape
    m_items = items.shape[0]
    t = n * l
    qt = jnp.transpose(queries, (1, 0, 2))
    grid = (n // _BLOCK_N,)
    rows_per_step = _BLOCK_N * l // 128
    out = pl.pallas_call(
        _fused_kernel,
        grid=grid,
        in_specs=[
            pl.BlockSpec((l, _BLOCK_N, c), lambda i: (0, i, 0)),
            pl.BlockSpec((m_items, c), lambda i: (0, 0)),
        ],
        out_specs=pl.BlockSpec((l, _BLOCK_N), lambda i: (0, i)),
        out_shape=jax.ShapeDtypeStruct((l, n), jnp.float32),
    )(qt, items)
    return out.T.reshape(t, 1)


# speedup vs baseline: 1.0494x; 1.0082x over previous
import jax
import jax.numpy as jnp
from jax.experimental import pallas as pl

_BLOCK_N = 256


def _fused_kernel(q_ref, items_ref, o_ref):
    s = jax.lax.dot_general(
        q_ref[...], items_ref[...],
        (((2,), (1,)), ((), ())),
        preferred_element_type=jnp.float32,
    )
    m = jnp.max(s, axis=2, keepdims=True)
    denom = jnp.sum(jnp.exp(s - m), axis=2)
    o_ref[...] = 1.0 / denom


@jax.jit
def kernel(queries, items):
    n, l, c = queries.shape
    m_items = items.shape[0]
    t = n * l
    qt = jnp.transpose(queries, (1, 0, 2))
    out = pl.pallas_call(
        _fused_kernel,
        grid=(n // _BLOCK_N,),
        in_specs=[
            pl.BlockSpec((l, _BLOCK_N, c), lambda i: (0, i, 0)),
            pl.BlockSpec((m_items, c), lambda i: (0, 0)),
        ],
        out_specs=pl.BlockSpec((l, _BLOCK_N), lambda i: (0, i)),
        out_shape=jax.ShapeDtypeStruct((l, n), jnp.float32),
    )(qt, items)
    return out.T.reshape(t, 1)
